# Initial kernel scaffold; baseline (speedup 1.0000x reference)
import jax
import jax.numpy as jnp
from jax.experimental import pallas as pl


def kernel(pos, x):
    raise NotImplementedError("write your pallas kernel here")


# speedup vs baseline: 25.3197x; 25.3197x over previous
import functools

import jax
import jax.numpy as jnp
from jax import lax
from jax.experimental import pallas as pl
from jax.experimental.pallas import tpu as pltpu
from jax.experimental.pallas import tpu_sc as plsc

B = 4
C_FEAT = 256
N = 16384
P = 4096
ROWS = N // 128


def _lane_iota(shape, dim):
    return lax.broadcasted_iota(jnp.int32, shape, dim)


def _extract_lane(rowvec, col):
    lane = _lane_iota((1, 128), 1)
    return jnp.sum(jnp.where(lane == col, rowvec, 0.0))


def _coords_at(pos_ref, b, idx):
    row = idx // 128
    col = idx - row * 128
    out = []
    for c in range(3):
        r = pos_ref[b, c, pl.ds(row, 1), :]
        out.append(_extract_lane(r, col))
    return tuple(out)


def _fps_body(pos_ref, idx_ref, dist_ref):
    lane1 = _lane_iota((1, 128), 1)
    iota2d = _lane_iota((128, 128), 0) * 128 + _lane_iota((128, 128), 1)

    init_coords = []
    for b in range(B):
        dist_ref[b] = jnp.full((128, 128), 1e10, dtype=jnp.float32)
        idx_ref[b] = jnp.zeros((32, 128), dtype=jnp.int32)
        init_coords.append(_coords_at(pos_ref, b, jnp.int32(0)))

    def body(i, carry):
        coords = carry
        row_i = i // 128
        col_i = i - row_i * 128
        new_coords = []
        for b in range(B):
            lx, ly, lz = coords[b]
            xs = pos_ref[b, 0]
            ys = pos_ref[b, 1]
            zs = pos_ref[b, 2]
            dx = xs - lx
            dy = ys - ly
            dz = zs - lz
            d = (dx * dx + dy * dy) + dz * dz
            dn = jnp.minimum(dist_ref[b], d)
            dist_ref[b] = dn
            m = jnp.max(dn)
            nxt = jnp.min(jnp.where(dn == m, iota2d, jnp.int32(N)))
            cur = idx_ref[b, pl.ds(row_i, 1), :]
            idx_ref[b, pl.ds(row_i, 1), :] = jnp.where(lane1 == col_i, nxt, cur)
            new_coords.append(_coords_at(pos_ref, b, nxt))
        return tuple(new_coords)

    lax.fori_loop(1, P, body, tuple(init_coords))


def _fps_indices(pos4):
    return pl.pallas_call(
        _fps_body,
        out_shape=jax.ShapeDtypeStruct((B, P // 128, 128), jnp.int32),
        scratch_shapes=[pltpu.VMEM((B, 128, 128), jnp.float32)],
    )(pos4)


def _make_gather_kernel():
    mesh = plsc.VectorSubcoreMesh(core_axis_name="c", subcore_axis_name="s")

    @functools.partial(
        pl.kernel,
        mesh=mesh,
        out_type=[
            jax.ShapeDtypeStruct((B, C_FEAT, P), jnp.float32),
            jax.ShapeDtypeStruct((B, 3, P), jnp.float32),
        ],
        scratch_types=[
            pltpu.VMEM((P,), jnp.int32),
            pltpu.VMEM((P,), jnp.float32),
            pltpu.SemaphoreType.DMA,
        ],
        compiler_params=pltpu.CompilerParams(use_tc_tiling_on_sc=False),
    )
    def gather_k(x_hbm, pos_hbm, idx_hbm, xout_hbm, pout_hbm, idx_v, out_v, sem):
        cid = lax.axis_index("c")
        sid = lax.axis_index("s")
        wid = sid * 2 + cid

        def do_channel(src_row, dst_row):
            pltpu.async_copy(src_row.at[idx_v], out_v, sem).wait()
            pltpu.sync_copy(out_v, dst_row)

        b = wid // 8
        grp = wid % 8
        pltpu.sync_copy(idx_hbm.at[b], idx_v)

        def ch_loop(k, _):
            ch = grp * 32 + k
            do_channel(x_hbm.at[b, ch], xout_hbm.at[b, ch])
            return 0

        lax.fori_loop(0, 32, ch_loop, 0)

        @pl.when(wid < 12)
        def _():
            b2 = wid // 3
            c2 = wid - b2 * 3
            pltpu.sync_copy(idx_hbm.at[b2], idx_v)
            do_channel(pos_hbm.at[b2, c2], pout_hbm.at[b2, c2])

    return gather_k


def kernel(pos, x):
    pos3 = pos[..., 0]
    x3 = x[..., 0]
    pos4 = pos3.reshape(B, 3, ROWS, 128)
    idx = _fps_indices(pos4).reshape(B, P)
    xg, pg = _make_gather_kernel()(x3, pos3, idx)
    return (xg[..., None], pg[..., None])


# speedup vs baseline: 65.5408x; 2.5885x over previous
import functools

import jax
import jax.numpy as jnp
from jax import lax
from jax.experimental import pallas as pl
from jax.experimental.pallas import tpu as pltpu
from jax.experimental.pallas import tpu_sc as plsc

B = 4
C_FEAT = 256
N = 16384
P = 4096
ROWS = N // 128


def _lane_iota(shape, dim):
    return lax.broadcasted_iota(jnp.int32, shape, dim)


def _fps_body(pos_ref, idx_ref, dist_ref):
    lane1 = _lane_iota((B, 1, 128), 2)
    iota3 = _lane_iota((B, 128, 128), 1) * 128 + _lane_iota((B, 128, 128), 2)

    def red2(v, op):
        return op(op(v, axis=1, keepdims=True), axis=2, keepdims=True)

    lanec = _lane_iota((1, 1, 128), 2)

    def pick(nxt):
        per_b = []
        for b in range(B):
            idxb = nxt[b, 0, 0]
            row = idxb // 128
            col = idxb - row * 128
            rows3 = pos_ref[b, :, pl.ds(row, 1), :]
            v = jnp.where(lanec == col, rows3, 0.0)
            v = jnp.sum(v, axis=2, keepdims=True)
            per_b.append(v)
        return jnp.stack(per_b, axis=0)

    dist_ref[...] = jnp.full((B, 128, 128), 1e10, dtype=jnp.float32)
    s0 = pick(jnp.zeros((B, 1, 1), dtype=jnp.int32))
    rowbuf0 = jnp.zeros((B, 1, 128), dtype=jnp.int32)

    def body(i, carry):
        s, rowbuf = carry
        lx, ly, lz = s[:, 0], s[:, 1], s[:, 2]
        row_i = i // 128
        col_i = i - row_i * 128
        pa = pos_ref[...]
        dx = pa[:, 0] - lx
        dy = pa[:, 1] - ly
        dz = pa[:, 2] - lz
        d = (dx * dx + dy * dy) + dz * dz
        dn = jnp.minimum(dist_ref[...], d)
        dist_ref[...] = dn
        m = red2(dn, jnp.max)
        cand = jnp.where(dn == m, iota3, jnp.int32(N))
        nxt = red2(cand, jnp.min)
        rowbuf = jnp.where(lane1 == col_i, nxt, rowbuf)
        idx_ref[:, pl.ds(row_i, 1), :] = rowbuf
        ns = pick(nxt)
        return (ns, rowbuf)

    lax.fori_loop(1, P, body, (s0, rowbuf0))


def _fps_indices(pos4):
    return pl.pallas_call(
        _fps_body,
        out_shape=jax.ShapeDtypeStruct((B, P // 128, 128), jnp.int32),
        scratch_shapes=[pltpu.VMEM((B, 128, 128), jnp.float32)],
    )(pos4)


def _make_gather_kernel():
    mesh = plsc.VectorSubcoreMesh(core_axis_name="c", subcore_axis_name="s")

    @functools.partial(
        pl.kernel,
        mesh=mesh,
        out_type=[
            jax.ShapeDtypeStruct((B, C_FEAT, P), jnp.float32),
            jax.ShapeDtypeStruct((B, 3, P), jnp.float32),
        ],
        scratch_types=[
            pltpu.VMEM((P,), jnp.int32),
            pltpu.VMEM((P,), jnp.float32),
            pltpu.SemaphoreType.DMA,
        ],
        compiler_params=pltpu.CompilerParams(use_tc_tiling_on_sc=False),
    )
    def gather_k(x_hbm, pos_hbm, idx_hbm, xout_hbm, pout_hbm, idx_v, out_v, sem):
        cid = lax.axis_index("c")
        sid = lax.axis_index("s")
        wid = sid * 2 + cid

        def do_channel(src_row, dst_row):
            pltpu.async_copy(src_row.at[idx_v], out_v, sem).wait()
            pltpu.sync_copy(out_v, dst_row)

        b = wid // 8
        grp = wid % 8
        pltpu.sync_copy(idx_hbm.at[b], idx_v)

        def ch_loop(k, _):
            ch = grp * 32 + k
            do_channel(x_hbm.at[b, ch], xout_hbm.at[b, ch])
            return 0

        lax.fori_loop(0, 32, ch_loop, 0)

        @pl.when(wid < 12)
        def _():
            b2 = wid // 3
            c2 = wid - b2 * 3
            pltpu.sync_copy(idx_hbm.at[b2], idx_v)
            do_channel(pos_hbm.at[b2, c2], pout_hbm.at[b2, c2])

    return gather_k


def kernel(pos, x):
    pos3 = pos[..., 0]
    x3 = x[..., 0]
    pos4 = pos3.reshape(B, 3, ROWS, 128)
    idx = _fps_indices(pos4).reshape(B, P)
    xg, pg = _make_gather_kernel()(x3, pos3, idx)
    return (xg[..., None], pg[..., None])


# speedup vs baseline: 73.6321x; 1.1235x over previous
import functools

import jax
import jax.numpy as jnp
from jax import lax
from jax.experimental import pallas as pl
from jax.experimental.pallas import tpu as pltpu
from jax.experimental.pallas import tpu_sc as plsc

B = 4
C_FEAT = 256
N = 16384
P = 4096
ROWS = N // 128


def _lane_iota(shape, dim):
    return lax.broadcasted_iota(jnp.int32, shape, dim)


def _fps_body(pos_ref, idx_ref, dist_ref):
    lane1 = _lane_iota((B, 1, 128), 2)
    iota3 = _lane_iota((B, 128, 128), 1) + 128 * _lane_iota((B, 128, 128), 2)
    BIG = jnp.int32(1 << 30)

    def merge(a, b):
        va, na = a[0], a[1]
        vb, nb = b[0], b[1]
        t = (vb > va) | ((vb == va) & (nb < na))
        return tuple(jnp.where(t, q, p) for p, q in zip(a, b))

    dist_ref[...] = jnp.full((B, 128, 128), 1e10, dtype=jnp.float32)
    lx0 = pos_ref[:, 0, 0:1, 0:1]
    ly0 = pos_ref[:, 1, 0:1, 0:1]
    lz0 = pos_ref[:, 2, 0:1, 0:1]
    rowbuf0 = jnp.zeros((B, 1, 128), dtype=jnp.int32)

    iota8 = _lane_iota((B, 8, 128), 1) + 128 * _lane_iota((B, 8, 128), 2)

    def body(i, carry):
        lxv, lyv, lzv, rowbuf = carry
        row_i = i // 128
        col_i = i - row_i * 128

        acc = None
        for r in range(16):
            sl = slice(8 * r, 8 * r + 8)
            xs = pos_ref[:, 0, sl, :]
            ys = pos_ref[:, 1, sl, :]
            zs = pos_ref[:, 2, sl, :]
            dx = xs - lxv
            dy = ys - lyv
            dz = zs - lzv
            d = (dx * dx + dy * dy) + dz * dz
            dn = jnp.minimum(dist_ref[:, sl, :], d)
            dist_ref[:, sl, :] = dn
            cand = (dn, iota8 + 8 * r, xs, ys, zs)
            acc = cand if acc is None else merge(acc, cand)

        for s in (4, 2, 1):
            rolled = tuple(pltpu.roll(f, s, axis=1) for f in acc)
            acc = merge(acc, rolled)
        cv, cn, cx, cy, cz = (f[:, :1] for f in acc)

        m = jnp.max(cv, axis=2, keepdims=True)

        mask = cv == m
        kn = jnp.where(mask, cn, BIG)
        pref = cn << 16

        def halves(c):
            bits = lax.bitcast_convert_type(c, jnp.int32)
            hi = jnp.where(mask, pref | (bits >> 16), BIG)
            lo = jnp.where(mask, pref | (bits & 0xFFFF), BIG)
            return hi, lo

        def redmin(k):
            return jnp.min(k, axis=2, keepdims=True)

        def unpack(hi, lo):
            bits = ((hi & 0xFFFF) << 16) | (lo & 0xFFFF)
            return lax.bitcast_convert_type(bits, jnp.float32)

        kxh, kxl = halves(cx)
        kyh, kyl = halves(cy)
        kzh, kzl = halves(cz)
        nxt = redmin(kn)
        nlx = unpack(redmin(kxh), redmin(kxl))
        nly = unpack(redmin(kyh), redmin(kyl))
        nlz = unpack(redmin(kzh), redmin(kzl))

        rowbuf = jnp.where(lane1 == col_i, nxt, rowbuf)
        idx_ref[:, pl.ds(row_i, 1), :] = rowbuf
        return (nlx, nly, nlz, rowbuf)

    lax.fori_loop(1, P, body, (lx0, ly0, lz0, rowbuf0))


def _fps_indices(pos4):
    return pl.pallas_call(
        _fps_body,
        out_shape=jax.ShapeDtypeStruct((B, P // 128, 128), jnp.int32),
        scratch_shapes=[pltpu.VMEM((B, 128, 128), jnp.float32)],
    )(pos4)


def _make_gather_kernel():
    mesh = plsc.VectorSubcoreMesh(core_axis_name="c", subcore_axis_name="s")

    @functools.partial(
        pl.kernel,
        mesh=mesh,
        out_type=[
            jax.ShapeDtypeStruct((B, C_FEAT, P), jnp.float32),
            jax.ShapeDtypeStruct((B, 3, P), jnp.float32),
        ],
        scratch_types=[
            pltpu.VMEM((P,), jnp.int32),
            pltpu.VMEM((P,), jnp.float32),
            pltpu.SemaphoreType.DMA,
        ],
        compiler_params=pltpu.CompilerParams(use_tc_tiling_on_sc=False),
    )
    def gather_k(x_hbm, pos_hbm, idx_hbm, xout_hbm, pout_hbm, idx_v, out_v, sem):
        cid = lax.axis_index("c")
        sid = lax.axis_index("s")
        wid = sid * 2 + cid

        def do_channel(src_row, dst_row):
            pltpu.async_copy(src_row.at[idx_v], out_v, sem).wait()
            pltpu.sync_copy(out_v, dst_row)

        b = wid // 8
        grp = wid % 8
        pltpu.sync_copy(idx_hbm.at[b], idx_v)

        def ch_loop(k, _):
            ch = grp * 32 + k
            do_channel(x_hbm.at[b, ch], xout_hbm.at[b, ch])
            return 0

        lax.fori_loop(0, 32, ch_loop, 0)

        @pl.when(wid < 12)
        def _():
            b2 = wid // 3
            c2 = wid - b2 * 3
            pltpu.sync_copy(idx_hbm.at[b2], idx_v)
            do_channel(pos_hbm.at[b2, c2], pout_hbm.at[b2, c2])

    return gather_k


def _prep_pos(pos3):
    return jnp.swapaxes(pos3.reshape(B, 3, ROWS, 128), 2, 3)


def kernel(pos, x):
    pos3 = pos[..., 0]
    x3 = x[..., 0]
    idx = _fps_indices(_prep_pos(pos3)).reshape(B, P)
    xg, pg = _make_gather_kernel()(x3, pos3, idx)
    return (xg[..., None], pg[..., None])


# speedup vs baseline: 78.7553x; 1.0696x over previous
import functools

import jax
import jax.numpy as jnp
from jax import lax
from jax.experimental import pallas as pl
from jax.experimental.pallas import tpu as pltpu
from jax.experimental.pallas import tpu_sc as plsc

B = 4
C_FEAT = 256
N = 16384
P = 4096
ROWS = N // 128


def _lane_iota(shape, dim):
    return lax.broadcasted_iota(jnp.int32, shape, dim)


def _fps_body(pos_ref, idx_ref, dist_ref):
    lane1 = _lane_iota((B, 1, 128), 2)
    BIG = jnp.int32(1 << 30)

    def merge(a, b):
        va, na = a[0], a[1]
        vb, nb = b[0], b[1]
        t = (vb > va) | ((vb == va) & (nb < na))
        return tuple(jnp.where(t, q, p) for p, q in zip(a, b))

    dist_ref[...] = jnp.full((B, 128, 128), 1e10, dtype=jnp.float32)
    lx0 = pos_ref[:, 0, 0:1, 0:1]
    ly0 = pos_ref[:, 1, 0:1, 0:1]
    lz0 = pos_ref[:, 2, 0:1, 0:1]
    rowbuf0 = jnp.zeros((B, 1, 128), dtype=jnp.int32)

    iota8 = _lane_iota((B, 8, 128), 1) + 128 * _lane_iota((B, 8, 128), 2)

    def body(i, carry):
        lxv, lyv, lzv, rowbuf = carry
        row_i = i // 128
        col_i = i - row_i * 128

        acc = None
        for r in range(16):
            sl = slice(8 * r, 8 * r + 8)
            xs = pos_ref[:, 0, sl, :]
            ys = pos_ref[:, 1, sl, :]
            zs = pos_ref[:, 2, sl, :]
            dx = xs - lxv
            dy = ys - lyv
            dz = zs - lzv
            d = (dx * dx + dy * dy) + dz * dz
            dn = jnp.minimum(dist_ref[:, sl, :], d)
            dist_ref[:, sl, :] = dn
            cand = (dn, iota8 + 8 * r, xs, ys, zs)
            if acc is None:
                acc = cand
            else:
                t = cand[0] > acc[0]
                acc = tuple(jnp.where(t, q, p) for p, q in zip(acc, cand))

        for s in (4, 2, 1):
            rolled = tuple(pltpu.roll(f, s, axis=1) for f in acc)
            acc = merge(acc, rolled)
        cv, cn, cx, cy, cz = (f[:, :1] for f in acc)

        m = jnp.max(cv, axis=2, keepdims=True)

        mask = cv == m
        kn = jnp.where(mask, cn, BIG)
        pref = cn << 16

        def halves(c):
            bits = lax.bitcast_convert_type(c, jnp.int32)
            hi = jnp.where(mask, pref | (bits >> 16), BIG)
            lo = jnp.where(mask, pref | (bits & 0xFFFF), BIG)
            return hi, lo

        def redmin(k):
            return jnp.min(k, axis=2, keepdims=True)

        def unpack(hi, lo):
            bits = ((hi & 0xFFFF) << 16) | (lo & 0xFFFF)
            return lax.bitcast_convert_type(bits, jnp.float32)

        kxh, kxl = halves(cx)
        kyh, kyl = halves(cy)
        kzh, kzl = halves(cz)
        nxt = redmin(kn)
        nlx = unpack(redmin(kxh), redmin(kxl))
        nly = unpack(redmin(kyh), redmin(kyl))
        nlz = unpack(redmin(kzh), redmin(kzl))

        rowbuf = jnp.where(lane1 == col_i, nxt, rowbuf)
        idx_ref[:, pl.ds(row_i, 1), :] = rowbuf
        return (nlx, nly, nlz, rowbuf)

    lax.fori_loop(1, P, body, (lx0, ly0, lz0, rowbuf0))


def _fps_indices(pos4):
    return pl.pallas_call(
        _fps_body,
        out_shape=jax.ShapeDtypeStruct((B, P // 128, 128), jnp.int32),
        scratch_shapes=[pltpu.VMEM((B, 128, 128), jnp.float32)],
    )(pos4)


def _make_gather_kernel():
    mesh = plsc.VectorSubcoreMesh(core_axis_name="c", subcore_axis_name="s")

    @functools.partial(
        pl.kernel,
        mesh=mesh,
        out_type=[
            jax.ShapeDtypeStruct((B, C_FEAT, P), jnp.float32),
            jax.ShapeDtypeStruct((B, 3, P), jnp.float32),
        ],
        scratch_types=[
            pltpu.VMEM((P,), jnp.int32),
            pltpu.VMEM((P,), jnp.float32),
            pltpu.VMEM((P,), jnp.float32),
            pltpu.SemaphoreType.DMA,
            pltpu.SemaphoreType.DMA,
            pltpu.SemaphoreType.DMA,
            pltpu.SemaphoreType.DMA,
        ],
        compiler_params=pltpu.CompilerParams(use_tc_tiling_on_sc=False),
    )
    def gather_k(x_hbm, pos_hbm, idx_hbm, xout_hbm, pout_hbm,
                 idx_v, out0, out1, gs0, gs1, ws0, ws1):
        cid = lax.axis_index("c")
        sid = lax.axis_index("s")
        wid = sid * 2 + cid

        b = wid // 8
        base = (wid % 8) * 32
        pltpu.sync_copy(idx_hbm.at[b], idx_v)

        outs = (out0, out1)
        gss = (gs0, gs1)
        wss = (ws0, ws1)

        pltpu.async_copy(x_hbm.at[b, base].at[idx_v], out0, gs0)
        pltpu.async_copy(x_hbm.at[b, base + 1].at[idx_v], out1, gs1)

        def step(j, _):
            for p in range(2):
                ch = base + 2 * j + p
                pltpu.make_async_copy(
                    x_hbm.at[b, ch].at[idx_v], outs[p], gss[p]
                ).wait()
                pltpu.async_copy(outs[p], xout_hbm.at[b, ch], wss[p])
            for p in range(2):
                k = 2 * j + p + 2

                @pl.when(k < 32)
                def _():
                    ch = base + k
                    pltpu.make_async_copy(
                        outs[p], xout_hbm.at[b, ch - 2], wss[p]
                    ).wait()
                    pltpu.async_copy(x_hbm.at[b, ch].at[idx_v], outs[p], gss[p])

            return 0

        lax.fori_loop(0, 16, step, 0)
        pltpu.make_async_copy(out0, xout_hbm.at[b, base + 30], ws0).wait()
        pltpu.make_async_copy(out1, xout_hbm.at[b, base + 31], ws1).wait()

        @pl.when(wid < 12)
        def _():
            b2 = wid // 3
            c2 = wid - b2 * 3
            pltpu.sync_copy(idx_hbm.at[b2], idx_v)
            pltpu.async_copy(pos_hbm.at[b2, c2].at[idx_v], out0, gs0).wait()
            pltpu.sync_copy(out0, pout_hbm.at[b2, c2])

    return gather_k


def _prep_pos(pos3):
    return jnp.swapaxes(pos3.reshape(B, 3, ROWS, 128), 2, 3)


def kernel(pos, x):
    pos3 = pos[..., 0]
    x3 = x[..., 0]
    idx = _fps_indices(_prep_pos(pos3)).reshape(B, P)
    xg, pg = _make_gather_kernel()(x3, pos3, idx)
    return (xg[..., None], pg[..., None])
